# baseline (speedup 1.0000x reference)
import functools

import jax
import jax.numpy as jnp
from jax import lax
from jax.experimental import pallas as pl
from jax.experimental.pallas import tpu as pltpu
from jax.experimental.pallas import tpu_sc as plsc

B = 16384
K = 16
NC = 2
NS = 16
NW = NC * NS
N = 1_000_000
LW = 1024
NLEAF = 977
LPT = 62
MAXCS = 999040
MCAP = 1536
L1CAP = 272
L2CAP = 64
BATCH = 128
DUMP = B

_mesh = plsc.VectorSubcoreMesh(core_axis_name="c", subcore_axis_name="s")

_scr_t = jax.ShapeDtypeStruct((B + 16, 128), jnp.float32)


@functools.partial(
    pl.kernel,
    out_type=[_scr_t, _scr_t],
    mesh=_mesh,
    scratch_types=[
        pltpu.VMEM((128, 128), jnp.int32),
        pltpu.VMEM((MCAP,), jnp.int32),
        pltpu.VMEM((MCAP,), jnp.int32),
        pltpu.VMEM((8 * L1CAP,), jnp.int32),
        pltpu.VMEM((8 * L1CAP,), jnp.int32),
        pltpu.VMEM((LPT * L2CAP,), jnp.int32),
        pltpu.VMEM((LPT * L2CAP,), jnp.int32),
        pltpu.VMEM((4 * K, LW), jnp.float32),
        pltpu.VMEM((BATCH, 128), jnp.float32),
        pltpu.VMEM((BATCH,), jnp.int32),
        pltpu.SemaphoreType.DMA,
    ],
    compiler_params=pltpu.CompilerParams(
        needs_layout_passes=False, use_tc_tiling_on_sc=True),
)
def _sweep(uid2d, iid2d, umat_t, imat_t, uscr, vscr,
           idxv, mu, mb, l1u, l1b, l2u, l2b, ring, batch, bidx, sem):
    core = lax.axis_index("c")
    tid = lax.axis_index("s")
    lane = lax.iota(jnp.int32, 16)
    lo = tid * LPT
    dump = jnp.full((16,), DUMP + tid, jnp.int32)
    sent = jnp.full((16,), -1, jnp.int32)

    def body(tab, idx2d, scr):
        pltpu.sync_copy(idx2d, idxv)
        def init_list(ref, n16, val):
            def w(i, _):
                ref[pl.ds(pl.multiple_of(i * 16, 16), 16)] = val
                return 0
            lax.fori_loop(0, n16, w, 0)
        init_list(mu, MCAP // 16, sent)
        init_list(l1u, 8 * L1CAP // 16, sent)
        init_list(l2u, LPT * L2CAP // 16, sent)
        init_list(bidx, BATCH // 16, dump)

        def scan_row(r, off):
            def scan_part(j, o):
                uv = idxv[r, pl.ds(j * 16, 16)]
                bv = r * 128 + j * 16 + lane
                lf = uv >> 10
                m = (lf >= lo) & (lf < lo + LPT)
                plsc.store_compressed(mu.at[pl.ds(o, 16)], uv, mask=m)
                plsc.store_compressed(mb.at[pl.ds(o, 16)], bv, mask=m)
                return o + plsc.all_reduce_population_count(m)[0]
            for j in range(8):
                off = scan_part(j, off)
            return off
        lax.fori_loop(0, 128, scan_row, 0)

        for p in range(8):
            def l1_pass(q, o, p=p):
                uv = mu[pl.ds(pl.multiple_of(q * 16, 16), 16)]
                bv = mb[pl.ds(pl.multiple_of(q * 16, 16), 16)]
                m = (uv >= 0) & (((uv >> 10) - lo) >> 3 == p)
                plsc.store_compressed(l1u.at[pl.ds(p * L1CAP + o, 16)], uv, mask=m)
                plsc.store_compressed(l1b.at[pl.ds(p * L1CAP + o, 16)], bv, mask=m)
                return o + plsc.all_reduce_population_count(m)[0]
            lax.fori_loop(0, MCAP // 16, functools.partial(l1_pass), 0)

        def l2_leaf(l, _):
            p = lax.div(l, 8)
            def l2_pass(q, o):
                base = p * L1CAP + q * 16
                uv = l1u[pl.ds(base, 16)]
                bv = l1b[pl.ds(base, 16)]
                m = (uv >= 0) & ((uv >> 10) == lo + l)
                plsc.store_compressed(l2u.at[pl.ds(l * L2CAP + o, 16)], uv, mask=m)
                plsc.store_compressed(l2b.at[pl.ds(l * L2CAP + o, 16)], bv, mask=m)
                return o + plsc.all_reduce_population_count(m)[0]
            lax.fori_loop(0, L1CAP // 16, l2_pass, 0)
            return 0
        lax.fori_loop(0, LPT, l2_leaf, 0)

        def colstart(l):
            return jnp.minimum((lo + l) * LW, MAXCS)

        def fire(l):
            cs = pl.multiple_of(colstart(l), 128)
            slot = pl.multiple_of(lax.rem(l, 4) * K, 8)
            pltpu.async_copy(
                tab.at[:, pl.ds(cs, LW)], ring.at[pl.ds(slot, K)], sem)

        for l in range(4):
            fire(l)

        def leaf(l, fill):
            pltpu.make_async_copy(
                tab.at[:, pl.ds(0, LW)], ring.at[pl.ds(0, K)], sem).wait()

            @pl.when(l + 4 < LPT)
            def _():
                fire(l + 4)

            slot = lax.rem(l, 4) * K
            cs = colstart(l)
            gl = lo + l
            for q in range(L2CAP // 16):
                uv = l2u[pl.ds(l * L2CAP + q * 16, 16)]
                bv = l2b[pl.ds(l * L2CAP + q * 16, 16)]
                m = (uv >> 10) == gl
                cols = jnp.maximum(uv - cs, 0)
                rows = fill + plsc.cumsum(jnp.where(m, 1, 0).astype(jnp.int32)) - 1
                for k in range(K):
                    vals = plsc.load_gather(
                        ring, [jnp.full((16,), slot + k, jnp.int32), cols], mask=m)
                    plsc.store_scatter(
                        batch, [rows, jnp.full((16,), k, jnp.int32)], vals, mask=m)
                plsc.store_scatter(bidx, [rows], bv, mask=m)
                fill = fill + plsc.all_reduce_population_count(m)[0]

            flush = fill >= BATCH - L2CAP

            @pl.when(flush)
            def _():
                pltpu.sync_copy(batch, scr.at[bidx])
                init_list(bidx, BATCH // 16, dump)

            return jnp.where(flush, 0, fill)

        lax.fori_loop(0, LPT, leaf, 0)
        pltpu.sync_copy(batch, scr.at[bidx])

    @pl.when(core == 0)
    def _():
        body(umat_t, uid2d, uscr)

    @pl.when(core == 1)
    def _():
        body(imat_t, iid2d, vscr)


@functools.partial(
    pl.kernel,
    out_type=jax.ShapeDtypeStruct((NW * 8, 128), jnp.float32),
    mesh=_mesh,
    scratch_types=[
        pltpu.VMEM((128, 128), jnp.float32),
        pltpu.VMEM((128, 128), jnp.float32),
        pltpu.VMEM((8, 128), jnp.float32),
        pltpu.SemaphoreType.DMA,
    ],
    compiler_params=pltpu.CompilerParams(
        needs_layout_passes=False, use_tc_tiling_on_sc=True),
)
def _dot(uscr, vscr, out, ubuf, vbuf, outv, sem):
    wid = lax.axis_index("s") * NC + lax.axis_index("c")
    lane = lax.iota(jnp.int32, 16)

    def chunk(ch, _):
        base = pl.multiple_of(wid * 512 + ch * 128, 128)
        pltpu.sync_copy(uscr.at[pl.ds(base, 128)], ubuf)
        pltpu.sync_copy(vscr.at[pl.ds(base, 128)], vbuf)
        for gg in range(8):
            rows = gg * 16 + lane
            acc = jnp.zeros((16,), jnp.float32)
            for k in range(K):
                col = jnp.full((16,), k, jnp.int32)
                acc = acc + (plsc.load_gather(ubuf, [rows, col])
                             * plsc.load_gather(vbuf, [rows, col]))
            outv[ch, pl.ds(gg * 16, 16)] = acc
        return 0

    lax.fori_loop(0, 4, chunk, 0)
    pltpu.sync_copy(outv, out.at[pl.ds(wid * 8, 8)])


def kernel(uid, iid, user_mat, item_mat):
    uid2d = uid.astype(jnp.int32).reshape((128, 128))
    iid2d = iid.astype(jnp.int32).reshape((128, 128))
    uscr, vscr = _sweep(uid2d, iid2d, user_mat.T, item_mat.T)
    padded = _dot(uscr, vscr)
    return padded.reshape(NW, 8, 128)[:, :4, :].reshape(B)


# speedup vs baseline: 1.7778x; 1.7778x over previous
import functools

import jax
import jax.numpy as jnp
from jax import lax
from jax.experimental import pallas as pl
from jax.experimental.pallas import tpu as pltpu
from jax.experimental.pallas import tpu_sc as plsc

B = 16384
K = 16
NC = 2
NS = 16
NW = NC * NS
BPW = B // NW
CH = 128
NCH = BPW // CH
G = BPW // 16
TBLK = 128

_mesh = plsc.VectorSubcoreMesh(core_axis_name="c", subcore_axis_name="s")


@functools.partial(
    pl.kernel,
    out_type=jax.ShapeDtypeStruct((NW * 8, 128), jnp.float32),
    mesh=_mesh,
    scratch_types=[
        pltpu.VMEM((NCH, CH), jnp.int32),
        pltpu.VMEM((NCH, CH), jnp.int32),
        pltpu.VMEM((16 * K, TBLK), jnp.float32),
        pltpu.VMEM((16 * K, TBLK), jnp.float32),
        pltpu.VMEM((BPW // 8, 128), jnp.float32),
        pltpu.VMEM((BPW // 8, 128), jnp.float32),
        pltpu.VMEM((8, 128), jnp.float32),
        pltpu.SemaphoreType.DMA,
    ],
    compiler_params=pltpu.CompilerParams(
        needs_layout_passes=False, use_tc_tiling_on_sc=True),
)
def _bpr_sc(uid2d, iid2d, umat_t, imat_t, out,
            uidx, iidx, ublk, vblk, urows, vrows, outv, sem):
    wid = lax.axis_index("s") * NC + lax.axis_index("c")
    pltpu.sync_copy(uid2d.at[pl.ds(wid * NCH, NCH)], uidx)
    pltpu.sync_copy(iid2d.at[pl.ds(wid * NCH, NCH)], iidx)

    lane = lax.iota(jnp.int32, 16)

    def idx_vecs(g):
        r = lax.div(g * 16, CH)
        c = lax.rem(g * 16, CH)
        return uidx[r, pl.ds(c, 16)], iidx[r, pl.ds(c, 16)]

    def fire(i, ustart_i, vstart_i):
        us = pl.multiple_of(ustart_i, TBLK)
        vs = pl.multiple_of(vstart_i, TBLK)
        for h in range(2):
            pltpu.async_copy(
                umat_t.at[pl.ds(h * 8, 8), pl.ds(us, TBLK)],
                ublk.at[pl.ds(i * K + h * 8, 8)], sem)
            pltpu.async_copy(
                imat_t.at[pl.ds(h * 8, 8), pl.ds(vs, TBLK)],
                vblk.at[pl.ds(i * K + h * 8, 8)], sem)

    uvec0, vvec0 = idx_vecs(0)
    ust0 = (uvec0 >> 7) * TBLK
    vst0 = (vvec0 >> 7) * TBLK
    for i in range(16):
        fire(i, ust0[i], vst0[i])

    def gather_group(g, _):
        uvec, vvec = idx_vecs(g)
        ucol = uvec & (TBLK - 1)
        vcol = vvec & (TBLK - 1)
        has_next = g + 1 < G
        nuvec, nvvec = idx_vecs(lax.rem(g + 1, G))
        nust = (nuvec >> 7) * TBLK
        nvst = (nvvec >> 7) * TBLK
        for i in range(16):
            for _h in range(2):
                pltpu.make_async_copy(
                    umat_t.at[pl.ds(0, 8), pl.ds(0, TBLK)],
                    ublk.at[pl.ds(0, 8)], sem).wait()
                pltpu.make_async_copy(
                    imat_t.at[pl.ds(0, 8), pl.ds(0, TBLK)],
                    vblk.at[pl.ds(0, 8)], sem).wait()
            uc = plsc.load_gather(
                ublk, [i * K + lane, jnp.full((16,), ucol[i], jnp.int32)])
            vc = plsc.load_gather(
                vblk, [i * K + lane, jnp.full((16,), vcol[i], jnp.int32)])
            urows[2 * g + i // 8, pl.ds((i % 8) * 16, 16)] = uc
            vrows[2 * g + i // 8, pl.ds((i % 8) * 16, 16)] = vc

            @pl.when(has_next)
            def _():
                fire(i, nust[i], nvst[i])

        return 0

    lax.fori_loop(0, G, gather_group, 0)

    def dot_group(g, _):
        rows = 2 * g + (lane >> 3)
        cols0 = (lane & 7) * 16
        acc = jnp.zeros((16,), jnp.float32)
        for k in range(K):
            uc = plsc.load_gather(urows, [rows, cols0 + k])
            vc = plsc.load_gather(vrows, [rows, cols0 + k])
            acc = acc + uc * vc
        outv[lax.div(g, 8), pl.ds(pl.multiple_of(lax.rem(g, 8) * 16, 8), 16)] = acc
        return 0

    lax.fori_loop(0, G, dot_group, 0)
    pltpu.sync_copy(outv, out.at[pl.ds(wid * 8, 8)])


def kernel(uid, iid, user_mat, item_mat):
    uid2d = uid.astype(jnp.int32).reshape((B // CH, CH))
    iid2d = iid.astype(jnp.int32).reshape((B // CH, CH))
    padded = _bpr_sc(uid2d, iid2d, user_mat.T, item_mat.T)
    return padded.reshape(NW, 8, 128)[:, :4, :].reshape(B)


# speedup vs baseline: 1.7903x; 1.0071x over previous
import functools

import jax
import jax.numpy as jnp
from jax import lax
from jax.experimental import pallas as pl
from jax.experimental.pallas import tpu as pltpu
from jax.experimental.pallas import tpu_sc as plsc

B = 16384
K = 16
NC = 2
NS = 16
NW = NC * NS
BPW = B // NW
CH = 128
NCH = BPW // CH
G = BPW // 16
TBLK = 128

_mesh = plsc.VectorSubcoreMesh(core_axis_name="c", subcore_axis_name="s")


@functools.partial(
    pl.kernel,
    out_type=jax.ShapeDtypeStruct((NW * 8, 128), jnp.float32),
    mesh=_mesh,
    scratch_types=[
        pltpu.VMEM((NCH, CH), jnp.int32),
        pltpu.VMEM((NCH, CH), jnp.int32),
        pltpu.VMEM((16 * K, TBLK), jnp.float32),
        pltpu.VMEM((16 * K, TBLK), jnp.float32),
        pltpu.VMEM((BPW // 8, 128), jnp.float32),
        pltpu.VMEM((BPW // 8, 128), jnp.float32),
        pltpu.VMEM((8, 128), jnp.float32),
        pltpu.SemaphoreType.DMA,
    ],
    compiler_params=pltpu.CompilerParams(
        needs_layout_passes=False, use_tc_tiling_on_sc=True),
)
def _bpr_sc(uid2d, iid2d, umat_t, imat_t, out,
            uidx, iidx, ublk, vblk, urows, vrows, outv, sem):
    wid = lax.axis_index("s") * NC + lax.axis_index("c")
    pltpu.sync_copy(uid2d.at[pl.ds(wid * NCH, NCH)], uidx)
    pltpu.sync_copy(iid2d.at[pl.ds(wid * NCH, NCH)], iidx)

    lane = lax.iota(jnp.int32, 16)

    def idx_vecs(g):
        r = lax.div(g * 16, CH)
        c = lax.rem(g * 16, CH)
        return uidx[r, pl.ds(c, 16)], iidx[r, pl.ds(c, 16)]

    def fire(i, ustart_i, vstart_i):
        us = pl.multiple_of(ustart_i, TBLK)
        vs = pl.multiple_of(vstart_i, TBLK)
        pltpu.async_copy(
            umat_t.at[:, pl.ds(us, TBLK)], ublk.at[pl.ds(i * K, K)], sem)
        pltpu.async_copy(
            imat_t.at[:, pl.ds(vs, TBLK)], vblk.at[pl.ds(i * K, K)], sem)

    uvec0, vvec0 = idx_vecs(0)
    ust0 = (uvec0 >> 7) * TBLK
    vst0 = (vvec0 >> 7) * TBLK
    for i in range(16):
        fire(i, ust0[i], vst0[i])

    def gather_group(g, _):
        uvec, vvec = idx_vecs(g)
        ucol = uvec & (TBLK - 1)
        vcol = vvec & (TBLK - 1)
        has_next = g + 1 < G
        nuvec, nvvec = idx_vecs(lax.rem(g + 1, G))
        nust = (nuvec >> 7) * TBLK
        nvst = (nvvec >> 7) * TBLK
        for i in range(16):
            pltpu.make_async_copy(
                umat_t.at[:, pl.ds(0, TBLK)], ublk.at[pl.ds(0, K)], sem).wait()
            pltpu.make_async_copy(
                imat_t.at[:, pl.ds(0, TBLK)], vblk.at[pl.ds(0, K)], sem).wait()
            uc = plsc.load_gather(
                ublk, [i * K + lane, jnp.full((16,), ucol[i], jnp.int32)])
            vc = plsc.load_gather(
                vblk, [i * K + lane, jnp.full((16,), vcol[i], jnp.int32)])
            urows[2 * g + i // 8, pl.ds((i % 8) * 16, 16)] = uc
            vrows[2 * g + i // 8, pl.ds((i % 8) * 16, 16)] = vc

            @pl.when(has_next)
            def _():
                fire(i, nust[i], nvst[i])

        return 0

    lax.fori_loop(0, G, gather_group, 0)

    def dot_group(g, _):
        rows = 2 * g + (lane >> 3)
        cols0 = (lane & 7) * 16
        acc = jnp.zeros((16,), jnp.float32)
        for k in range(K):
            uc = plsc.load_gather(urows, [rows, cols0 + k])
            vc = plsc.load_gather(vrows, [rows, cols0 + k])
            acc = acc + uc * vc
        outv[lax.div(g, 8), pl.ds(pl.multiple_of(lax.rem(g, 8) * 16, 8), 16)] = acc
        return 0

    lax.fori_loop(0, G, dot_group, 0)
    pltpu.sync_copy(outv, out.at[pl.ds(wid * 8, 8)])


def kernel(uid, iid, user_mat, item_mat):
    uid2d = uid.astype(jnp.int32).reshape((B // CH, CH))
    iid2d = iid.astype(jnp.int32).reshape((B // CH, CH))
    padded = _bpr_sc(uid2d, iid2d, user_mat.T, item_mat.T)
    return padded.reshape(NW, 8, 128)[:, :4, :].reshape(B)
